# Initial kernel scaffold; baseline (speedup 1.0000x reference)
import jax
import jax.numpy as jnp
from jax import lax
from jax.experimental import pallas as pl
from jax.experimental.pallas import tpu as pltpu
from jax.experimental.pallas import tpu_sc as plsc

NUM_EMBEDDINGS = 1000000
EMBEDDING_DIM = 32
BATCH = 4096
SEQ_LEN = 200

_B = BATCH * SEQ_LEN
_NC = 2
_NS = 16
_NW = _NC * _NS
_PER_W = _B // _NW
_CHUNK = 1024
_NCHUNK = _PER_W // _CHUNK


def _body(x_hbm, w_hbm, out_hbm, idx_v, rows_v, gsem):
    wid = lax.axis_index("s") * _NC + lax.axis_index("c")
    base = wid * _PER_W

    def chunk(k, carry):
        off = base + k * _CHUNK
        pltpu.sync_copy(x_hbm.at[pl.ds(off, _CHUNK)], idx_v)
        pltpu.async_copy(w_hbm.at[idx_v], rows_v, gsem).wait()
        pltpu.sync_copy(rows_v, out_hbm.at[pl.ds(off, _CHUNK)])
        return carry

    lax.fori_loop(0, _NCHUNK, chunk, 0)


@jax.jit
def _run(x_flat, weight):
    mesh = plsc.VectorSubcoreMesh(core_axis_name="c", subcore_axis_name="s")
    return pl.kernel(
        _body,
        out_type=jax.ShapeDtypeStruct((_B, EMBEDDING_DIM), jnp.float32),
        mesh=mesh,
        scratch_types=[
            pltpu.VMEM((_CHUNK,), jnp.int32),
            pltpu.VMEM((_CHUNK, EMBEDDING_DIM), jnp.float32),
            pltpu.SemaphoreType.DMA,
        ],
    )(x_flat, weight)


def kernel(x, weight):
    out = _run(x.reshape(-1), weight)
    return out.reshape(x.shape[0], x.shape[1], EMBEDDING_DIM)


# speedup vs baseline: 1.4620x; 1.4620x over previous
import jax
import jax.numpy as jnp
from jax import lax
from jax.experimental import pallas as pl
from jax.experimental.pallas import tpu as pltpu
from jax.experimental.pallas import tpu_sc as plsc

NUM_EMBEDDINGS = 1000000
EMBEDDING_DIM = 32
BATCH = 4096
SEQ_LEN = 200

_B = BATCH * SEQ_LEN
_NC = 2
_NS = 16
_NW = _NC * _NS
_PER_W = _B // _NW
_CHUNK = 1024
_NCHUNK = _PER_W // _CHUNK


def _body(x_hbm, w_hbm, out_hbm, idx_v, rows_v, gsem):
    wid = lax.axis_index("s") * _NC + lax.axis_index("c")
    base = wid * _PER_W

    def chunk(k, carry):
        off = base + k * _CHUNK
        pltpu.sync_copy(x_hbm.at[pl.ds(off, _CHUNK)], idx_v)
        pltpu.async_copy(w_hbm.at[idx_v], rows_v, gsem).wait()
        pltpu.sync_copy(rows_v, out_hbm.at[pl.ds(off, _CHUNK)])
        return carry

    lax.fori_loop(0, _NCHUNK, chunk, 0)


@jax.jit
def _run(x_flat, weight):
    mesh = plsc.VectorSubcoreMesh(core_axis_name="c", subcore_axis_name="s")
    return pl.kernel(
        _body,
        out_type=jax.ShapeDtypeStruct((_B, EMBEDDING_DIM), jnp.float32),
        mesh=mesh,
        scratch_types=[
            pltpu.VMEM((_CHUNK,), jnp.int32),
            pltpu.VMEM((_CHUNK, EMBEDDING_DIM), jnp.float32),
            pltpu.SemaphoreType.DMA,
        ],
        compiler_params=pltpu.CompilerParams(use_tc_tiling_on_sc=False),
    )(x_flat, weight)


def kernel(x, weight):
    out = _run(x.reshape(-1), weight)
    return out.reshape(x.shape[0], x.shape[1], EMBEDDING_DIM)


# speedup vs baseline: 1.5030x; 1.0280x over previous
import jax
import jax.numpy as jnp
from jax import lax
from jax.experimental import pallas as pl
from jax.experimental.pallas import tpu as pltpu
from jax.experimental.pallas import tpu_sc as plsc

NUM_EMBEDDINGS = 1000000
EMBEDDING_DIM = 32
BATCH = 4096
SEQ_LEN = 200

_B = BATCH * SEQ_LEN
_NC = 2
_NS = 16
_NW = _NC * _NS
_PER_W = _B // _NW
_CHUNK = 1600
_NCHUNK = _PER_W // _CHUNK


def _body(x_hbm, w_hbm, out_hbm, idx_v, rows_v, gsems, wsems):
    wid = lax.axis_index("s") * _NC + lax.axis_index("c")
    base = wid * _PER_W

    def start_gather(k, b):
        pltpu.make_async_copy(
            w_hbm.at[idx_v.at[pl.ds(k * _CHUNK, _CHUNK)]],
            rows_v.at[b],
            gsems.at[b],
        ).start()

    def wait_gather(b):
        pltpu.make_async_copy(
            w_hbm.at[idx_v.at[pl.ds(0, _CHUNK)]], rows_v.at[b], gsems.at[b]
        ).wait()

    def start_write(k, b):
        pltpu.make_async_copy(
            rows_v.at[b],
            out_hbm.at[pl.ds(base + k * _CHUNK, _CHUNK)],
            wsems.at[b],
        ).start()

    def wait_write(b):
        pltpu.make_async_copy(
            rows_v.at[b], out_hbm.at[pl.ds(base, _CHUNK)], wsems.at[b]
        ).wait()

    pltpu.sync_copy(x_hbm.at[pl.ds(base, _PER_W)], idx_v)

    start_gather(0, 0)
    start_gather(1, 1)
    wait_gather(0)
    start_write(0, 0)

    def steady(p, carry):
        for b in range(2):
            k = 2 * p + b
            wait_write(b)
            start_gather(k, b)
            wait_gather(1 - b)
            start_write(k - 1, 1 - b)
        return carry

    lax.fori_loop(1, _NCHUNK // 2, steady, 0)

    wait_gather(1)
    start_write(_NCHUNK - 1, 1)
    wait_write(0)
    wait_write(1)


@jax.jit
def _run(x_flat, weight):
    mesh = plsc.VectorSubcoreMesh(core_axis_name="c", subcore_axis_name="s")
    return pl.kernel(
        _body,
        out_type=jax.ShapeDtypeStruct((_B, EMBEDDING_DIM), jnp.float32),
        mesh=mesh,
        scratch_types=[
            pltpu.VMEM((_PER_W,), jnp.int32),
            pltpu.VMEM((2, _CHUNK, EMBEDDING_DIM), jnp.float32),
            pltpu.SemaphoreType.DMA((2,)),
            pltpu.SemaphoreType.DMA((2,)),
        ],
        compiler_params=pltpu.CompilerParams(use_tc_tiling_on_sc=False),
    )(x_flat, weight)


def kernel(x, weight):
    out = _run(x.reshape(-1), weight)
    return out.reshape(x.shape[0], x.shape[1], EMBEDDING_DIM)


# speedup vs baseline: 1.5036x; 1.0004x over previous
import jax
import jax.numpy as jnp
from jax import lax
from jax.experimental import pallas as pl
from jax.experimental.pallas import tpu as pltpu
from jax.experimental.pallas import tpu_sc as plsc

NUM_EMBEDDINGS = 1000000
EMBEDDING_DIM = 32
BATCH = 4096
SEQ_LEN = 200

_B = BATCH * SEQ_LEN
_NC = 2
_NS = 16
_NW = _NC * _NS
_PER_W = _B // _NW
_CHUNK = 1280
_NCHUNK = _PER_W // _CHUNK
_SUB = 8
_SUBROWS = _CHUNK // _SUB


def _body(x_hbm, w_hbm, out_hbm, idx_v, rows_v, gsems, wsems):
    wid = lax.axis_index("s") * _NC + lax.axis_index("c")
    base = wid * _PER_W

    def start_gather(k, b):
        for j in range(_SUB):
            pltpu.make_async_copy(
                w_hbm.at[idx_v.at[pl.ds(k * _CHUNK + j * _SUBROWS, _SUBROWS)]],
                rows_v.at[b].at[pl.ds(j * _SUBROWS, _SUBROWS)],
                gsems.at[b],
            ).start()

    def wait_gather(b):
        pltpu.make_async_copy(
            w_hbm.at[idx_v.at[pl.ds(0, _CHUNK)]], rows_v.at[b], gsems.at[b]
        ).wait()

    def start_write(k, b):
        pltpu.make_async_copy(
            rows_v.at[b],
            out_hbm.at[pl.ds(base + k * _CHUNK, _CHUNK)],
            wsems.at[b],
        ).start()

    def wait_write(b):
        pltpu.make_async_copy(
            rows_v.at[b], out_hbm.at[pl.ds(base, _CHUNK)], wsems.at[b]
        ).wait()

    pltpu.sync_copy(x_hbm.at[pl.ds(base, _PER_W)], idx_v)

    start_gather(0, 0)
    start_gather(1, 1)
    wait_gather(0)
    start_write(0, 0)

    def steady(p, carry):
        for b in range(2):
            k = 2 * p + b
            wait_write(b)
            start_gather(k, b)
            wait_gather(1 - b)
            start_write(k - 1, 1 - b)
        return carry

    lax.fori_loop(1, _NCHUNK // 2, steady, 0)

    wait_gather(1)
    start_write(_NCHUNK - 1, 1)
    wait_write(0)
    wait_write(1)


@jax.jit
def _run(x_flat, weight):
    mesh = plsc.VectorSubcoreMesh(core_axis_name="c", subcore_axis_name="s")
    return pl.kernel(
        _body,
        out_type=jax.ShapeDtypeStruct((_B, EMBEDDING_DIM), jnp.float32),
        mesh=mesh,
        scratch_types=[
            pltpu.VMEM((_PER_W,), jnp.int32),
            pltpu.VMEM((2, _CHUNK, EMBEDDING_DIM), jnp.float32),
            pltpu.SemaphoreType.DMA((2,)),
            pltpu.SemaphoreType.DMA((2,)),
        ],
        compiler_params=pltpu.CompilerParams(use_tc_tiling_on_sc=False),
    )(x_flat, weight)


def kernel(x, weight):
    out = _run(x.reshape(-1), weight)
    return out.reshape(x.shape[0], x.shape[1], EMBEDDING_DIM)
